# Initial kernel scaffold; baseline (speedup 1.0000x reference)
import jax
import jax.numpy as jnp
from jax.experimental import pallas as pl


def kernel(neighbors0, neighbors1, neighbors2, user_feat_idx, item_feat_idx, user_feat_emb, item_feat_emb, user_proj_w, user_proj_b, item_proj_w, item_proj_b, w0_w, w0_b, w1_w, w1_b):
    raise NotImplementedError("write your pallas kernel here")


# speedup vs baseline: 9.9350x; 9.9350x over previous
import functools

import jax
import jax.numpy as jnp
from jax import lax
from jax.experimental import pallas as pl
from jax.experimental.pallas import tpu as pltpu
from jax.experimental.pallas import tpu_sc as plsc

D = 64
NC, NS = 2, 16
NW = NC * NS
F = 8
B = 2048


def _wid():
    return lax.axis_index("s") * NC + lax.axis_index("c")


def _make_embag(n_pad, u_chunk):
    per_w = n_pad // NW
    n_chunks = per_w // u_chunk
    assert per_w % u_chunk == 0

    @functools.partial(
        pl.kernel,
        out_type=jax.ShapeDtypeStruct((n_pad, D), jnp.float32),
        mesh=plsc.VectorSubcoreMesh(core_axis_name="c", subcore_axis_name="s"),
        compiler_params=pltpu.CompilerParams(use_tc_tiling_on_sc=False),
        scratch_types=[
            pltpu.VMEM((u_chunk * F,), jnp.int32),
            pltpu.VMEM((u_chunk * F, D), jnp.float32),
            pltpu.VMEM((u_chunk, D), jnp.float32),
            pltpu.SemaphoreType.DMA,
        ],
    )
    def embag(table_hbm, idx_hbm, out_hbm, idx_v, rows_v, acc_v, sem):
        row0 = _wid() * per_w

        def chunk_body(c, carry):
            base = row0 + c * u_chunk
            pltpu.sync_copy(idx_hbm.at[pl.ds(base * F, u_chunk * F)], idx_v)
            pltpu.async_copy(table_hbm.at[idx_v], rows_v, sem).wait()

            def user_body(u, carry2):
                r = u * F
                for j in range(D // 16):
                    s = rows_v[r, pl.ds(16 * j, 16)]
                    for f in range(1, F):
                        s = s + rows_v[r + f, pl.ds(16 * j, 16)]
                    acc_v[u, pl.ds(16 * j, 16)] = s
                return carry2

            lax.fori_loop(0, u_chunk, user_body, 0)
            pltpu.sync_copy(acc_v, out_hbm.at[pl.ds(base, u_chunk)])
            return carry

        lax.fori_loop(0, n_chunks, chunk_body, 0)

    return embag


def _make_segsum(group, roots_per_chunk):
    per_w = B // NW
    n_chunks = per_w // roots_per_chunk
    assert per_w % roots_per_chunk == 0
    rows_per_chunk = roots_per_chunk * group

    @functools.partial(
        pl.kernel,
        out_type=jax.ShapeDtypeStruct((B, D), jnp.float32),
        mesh=plsc.VectorSubcoreMesh(core_axis_name="c", subcore_axis_name="s"),
        compiler_params=pltpu.CompilerParams(use_tc_tiling_on_sc=False),
        scratch_types=[
            pltpu.VMEM((rows_per_chunk,), jnp.int32),
            pltpu.VMEM((rows_per_chunk, D), jnp.float32),
            pltpu.VMEM((per_w, D), jnp.float32),
            pltpu.SemaphoreType.DMA,
        ],
    )
    def segsum(table_hbm, idx_hbm, out_hbm, idx_v, rows_v, acc_v, sem):
        root0 = _wid() * per_w

        def chunk_body(c, carry):
            rbase = root0 + c * roots_per_chunk
            pltpu.sync_copy(idx_hbm.at[pl.ds(rbase * group, rows_per_chunk)], idx_v)
            if group == 1:
                pltpu.async_copy(table_hbm.at[idx_v], acc_v, sem).wait()
                return carry
            pltpu.async_copy(table_hbm.at[idx_v], rows_v, sem).wait()

            def root_body(t, carry2):
                lo = t * group

                def add_body(k, ss):
                    return tuple(
                        ss[j] + rows_v[lo + k, pl.ds(16 * j, 16)]
                        for j in range(D // 16)
                    )

                init = tuple(rows_v[lo, pl.ds(16 * j, 16)] for j in range(D // 16))
                ss = lax.fori_loop(1, group, add_body, init, unroll=4)
                for j in range(D // 16):
                    acc_v[c * roots_per_chunk + t, pl.ds(16 * j, 16)] = ss[j]
                return carry2

            lax.fori_loop(0, roots_per_chunk, root_body, 0)
            return carry

        lax.fori_loop(0, n_chunks, chunk_body, 0)
        pltpu.sync_copy(acc_v, out_hbm.at[pl.ds(root0, per_w)])

    return segsum


_NU_PAD = 100352
_NI_PAD = 50176

_embag_users = _make_embag(_NU_PAD, 112)
_embag_items = _make_embag(_NI_PAD, 112)
_seg256 = _make_segsum(256, 4)
_seg16 = _make_segsum(16, 64)
_seg1 = _make_segsum(1, 64)


def _finale(mu0_ref, mi1_ref, mu2_ref, wu_ref, bu_ref, wi_ref, bi_ref,
            w0_ref, b0_ref, w1_ref, b1_ref, out_ref):
    def dot_t(a, w):
        return lax.dot_general(a, w, (((1,), (1,)), ((), ())),
                               precision=lax.Precision.HIGHEST,
                               preferred_element_type=jnp.float32)

    inv_f = 1.0 / F
    h0 = dot_t(mu0_ref[...] * inv_f, wu_ref[...]) + bu_ref[...]
    h1s = dot_t(mi1_ref[...] * inv_f, wi_ref[...]) + 16.0 * bi_ref[...]
    s2s = dot_t(mu2_ref[...] * inv_f, wu_ref[...]) + 256.0 * bu_ref[...]
    x1 = dot_t(jnp.concatenate([h0, h1s], axis=1), w0_ref[...]) + b0_ref[...]
    agg = dot_t(jnp.concatenate([h1s, s2s], axis=1), w0_ref[...]) + 16.0 * b0_ref[...]
    out_ref[...] = dot_t(jnp.concatenate([x1, agg], axis=1), w1_ref[...]) + b1_ref[...]


def kernel(neighbors0, neighbors1, neighbors2, user_feat_idx, item_feat_idx,
           user_feat_emb, item_feat_emb, user_proj_w, user_proj_b,
           item_proj_w, item_proj_b, w0_w, w0_b, w1_w, w1_b):
    i32 = jnp.int32
    f32 = jnp.float32
    ufi = jnp.concatenate([
        user_feat_idx.astype(i32).reshape(-1),
        jnp.zeros(((_NU_PAD - user_feat_idx.shape[0]) * F,), i32),
    ])
    ifi = jnp.concatenate([
        item_feat_idx.astype(i32).reshape(-1),
        jnp.zeros(((_NI_PAD - item_feat_idx.shape[0]) * F,), i32),
    ])
    su = _embag_users(user_feat_emb.astype(f32), ufi)
    si = _embag_items(item_feat_emb.astype(f32), ifi)
    mu0 = _seg1(su, neighbors0.astype(i32))
    mi1 = _seg16(si, neighbors1.astype(i32))
    mu2 = _seg256(su, neighbors2.astype(i32))

    row = lambda v: v.astype(f32).reshape(1, D)
    return pl.pallas_call(
        _finale,
        out_shape=jax.ShapeDtypeStruct((B, D), jnp.float32),
    )(mu0, mi1, mu2,
      user_proj_w.astype(f32), row(user_proj_b),
      item_proj_w.astype(f32), row(item_proj_b),
      w0_w.astype(f32), row(w0_b),
      w1_w.astype(f32), row(w1_b))


# speedup vs baseline: 12.7057x; 1.2789x over previous
import functools

import jax
import jax.numpy as jnp
from jax import lax
from jax.experimental import pallas as pl
from jax.experimental.pallas import tpu as pltpu
from jax.experimental.pallas import tpu_sc as plsc

D = 64
NJ = D // 16
NC, NS = 2, 16
NW = NC * NS
F = 8
B = 2048

_NU_PAD = 100352
_NI_PAD = 50176
_UPW = _NU_PAD // NW
_IPW = _NI_PAD // NW
_CH = 56
_UCHUNKS = _UPW // _CH
_ICHUNKS = _IPW // _CH


def _wid():
    return lax.axis_index("s") * NC + lax.axis_index("c")


def _embag_stage(table, idx_v, out_hbm, row0, n_chunks, rows_bufs, gsems,
                 accs, ssem):
    rpc = _CH * F

    def issue(c, rb, gs):
        pltpu.make_async_copy(
            table.at[idx_v.at[pl.ds(c * rpc, rpc)]], rb, gs).start()

    def wait_gather(rb, gs):
        pltpu.make_async_copy(
            table.at[idx_v.at[pl.ds(0, rpc)]], rb, gs).wait()

    def wait_store():
        pltpu.make_async_copy(
            accs[0], out_hbm.at[pl.ds(row0, _CH)], ssem).wait()

    issue(0, rows_bufs[0], gsems[0])
    issue(1, rows_bufs[1], gsems[1])

    def quad(k, carry):
        @pl.when(k >= 1)
        def _():
            for _unused in range(4):
                wait_store()

        c0 = k * 4
        for q in range(4):
            c = c0 + q
            rb, gs, ab = rows_bufs[q % 2], gsems[q % 2], accs[q]
            wait_gather(rb, gs)

            def user_body(u, carry2):
                r = u * F
                for j in range(NJ):
                    s = rb[r, pl.ds(16 * j, 16)]
                    for f in range(1, F):
                        s = s + rb[r + f, pl.ds(16 * j, 16)]
                    ab[u, pl.ds(16 * j, 16)] = s
                return carry2

            lax.fori_loop(0, _CH, user_body, 0)

            @pl.when(c + 2 < n_chunks)
            def _():
                issue(c + 2, rb, gs)

            pltpu.make_async_copy(
                ab, out_hbm.at[pl.ds(row0 + c * _CH, _CH)], ssem).start()
        return carry

    lax.fori_loop(0, n_chunks // 4, quad, 0)
    for _unused in range(4):
        wait_store()


@functools.partial(
    pl.kernel,
    out_type=(jax.ShapeDtypeStruct((_NU_PAD, D), jnp.float32),
              jax.ShapeDtypeStruct((_NI_PAD, D), jnp.float32)),
    mesh=plsc.VectorSubcoreMesh(core_axis_name="c", subcore_axis_name="s"),
    compiler_params=pltpu.CompilerParams(use_tc_tiling_on_sc=False),
    scratch_types=[
        pltpu.VMEM((_UPW * F,), jnp.int32),
        pltpu.VMEM((_IPW * F,), jnp.int32),
        pltpu.VMEM((_CH * F, D), jnp.float32),
        pltpu.VMEM((_CH * F, D), jnp.float32),
        pltpu.VMEM((_CH, D), jnp.float32),
        pltpu.VMEM((_CH, D), jnp.float32),
        pltpu.VMEM((_CH, D), jnp.float32),
        pltpu.VMEM((_CH, D), jnp.float32),
        pltpu.SemaphoreType.DMA,
        pltpu.SemaphoreType.DMA,
        pltpu.SemaphoreType.DMA,
    ],
)
def _embag2(ue_hbm, ufi_hbm, ie_hbm, ifi_hbm, su_hbm, si_hbm,
            uidx_v, iidx_v, rows0, rows1, acc0, acc1, acc2, acc3,
            gsem0, gsem1, ssem):
    wid = _wid()
    u0 = wid * _UPW
    i0 = wid * _IPW
    pltpu.sync_copy(ufi_hbm.at[pl.ds(u0 * F, _UPW * F)], uidx_v)
    pltpu.sync_copy(ifi_hbm.at[pl.ds(i0 * F, _IPW * F)], iidx_v)
    rows_bufs = (rows0, rows1)
    gsems = (gsem0, gsem1)
    accs = (acc0, acc1, acc2, acc3)
    _embag_stage(ue_hbm, uidx_v, su_hbm, u0, _UCHUNKS, rows_bufs, gsems,
                 accs, ssem)
    _embag_stage(ie_hbm, iidx_v, si_hbm, i0, _ICHUNKS, rows_bufs, gsems,
                 accs, ssem)


_RPW = B // NW
_G2 = 256
_G1 = 16
_SEG_ROWS = 512
_N2CH = _RPW * _G2 // _SEG_ROWS


@functools.partial(
    pl.kernel,
    out_type=(jax.ShapeDtypeStruct((B, D), jnp.float32),
              jax.ShapeDtypeStruct((B, D), jnp.float32),
              jax.ShapeDtypeStruct((B, D), jnp.float32)),
    mesh=plsc.VectorSubcoreMesh(core_axis_name="c", subcore_axis_name="s"),
    compiler_params=pltpu.CompilerParams(use_tc_tiling_on_sc=False),
    scratch_types=[
        pltpu.VMEM((_RPW * _G2,), jnp.int32),
        pltpu.VMEM((_RPW * _G1,), jnp.int32),
        pltpu.VMEM((_RPW,), jnp.int32),
        pltpu.VMEM((_SEG_ROWS, D), jnp.float32),
        pltpu.VMEM((_SEG_ROWS, D), jnp.float32),
        pltpu.VMEM((_RPW, D), jnp.float32),
        pltpu.VMEM((_RPW, D), jnp.float32),
        pltpu.VMEM((_RPW, D), jnp.float32),
        pltpu.SemaphoreType.DMA,
        pltpu.SemaphoreType.DMA,
        pltpu.SemaphoreType.DMA,
    ],
)
def _segsum3(su_hbm, si_hbm, n0_hbm, n1_hbm, n2_hbm,
             mu0_hbm, mi1_hbm, mu2_hbm,
             idx2_v, idx1_v, idx0_v, rows0, rows1, acc2, acc1, acc0,
             gsem0, gsem1, gsem2):
    wid = _wid()
    r0 = wid * _RPW
    pltpu.sync_copy(n0_hbm.at[pl.ds(r0, _RPW)], idx0_v)
    seg1 = pltpu.make_async_copy(su_hbm.at[idx0_v], acc0, gsem2)
    seg1.start()
    pltpu.sync_copy(n2_hbm.at[pl.ds(r0 * _G2, _RPW * _G2)], idx2_v)
    pltpu.sync_copy(n1_hbm.at[pl.ds(r0 * _G1, _RPW * _G1)], idx1_v)

    rows_bufs = (rows0, rows1)
    gsems = (gsem0, gsem1)

    def issue2(c, rb, gs):
        pltpu.make_async_copy(
            su_hbm.at[idx2_v.at[pl.ds(c * _SEG_ROWS, _SEG_ROWS)]], rb,
            gs).start()

    def wait_gather(table, rb, gs):
        pltpu.make_async_copy(
            table.at[idx2_v.at[pl.ds(0, _SEG_ROWS)]], rb, gs).wait()

    def reduce_group(rb, lo, group, acc, slot):
        def add_body(k, ss):
            return tuple(ss[j] + rb[lo + k, pl.ds(16 * j, 16)]
                         for j in range(NJ))

        init = tuple(rb[lo, pl.ds(16 * j, 16)] for j in range(NJ))
        ss = lax.fori_loop(1, group, add_body, init, unroll=4)
        for j in range(NJ):
            acc[slot, pl.ds(16 * j, 16)] = ss[j]

    issue2(0, rows0, gsem0)
    issue2(1, rows1, gsem1)

    def quad2(k, carry):
        c0 = k * 4
        for q in range(4):
            c = c0 + q
            rb, gs = rows_bufs[q % 2], gsems[q % 2]
            wait_gather(su_hbm, rb, gs)
            for lt in range(2):
                reduce_group(rb, lt * _G2, _G2, acc2, c * 2 + lt)

            @pl.when(c + 2 < _N2CH)
            def _():
                issue2(c + 2, rb, gs)
        return carry

    lax.fori_loop(0, _N2CH // 4, quad2, 0)

    for h in range(2):
        pltpu.make_async_copy(
            si_hbm.at[idx1_v.at[pl.ds(h * _SEG_ROWS, _SEG_ROWS)]],
            rows_bufs[h], gsems[h]).start()
    for h in range(2):
        wait_gather(si_hbm, rows_bufs[h], gsems[h])

        def root_body(t, carry2):
            reduce_group(rows_bufs[h], t * _G1, _G1, acc1,
                         h * (_SEG_ROWS // _G1) + t)
            return carry2

        lax.fori_loop(0, _SEG_ROWS // _G1, root_body, 0)

    pltpu.sync_copy(acc2, mu2_hbm.at[pl.ds(r0, _RPW)])
    pltpu.sync_copy(acc1, mi1_hbm.at[pl.ds(r0, _RPW)])
    seg1.wait()
    pltpu.sync_copy(acc0, mu0_hbm.at[pl.ds(r0, _RPW)])


def _finale(mu0_ref, mi1_ref, mu2_ref, wu_ref, bu_ref, wi_ref, bi_ref,
            w0_ref, b0_ref, w1_ref, b1_ref, out_ref):
    def dot_t(a, w):
        return lax.dot_general(a, w, (((1,), (1,)), ((), ())),
                               precision=lax.Precision.HIGHEST,
                               preferred_element_type=jnp.float32)

    inv_f = 1.0 / F
    h0 = dot_t(mu0_ref[...] * inv_f, wu_ref[...]) + bu_ref[...]
    h1s = dot_t(mi1_ref[...] * inv_f, wi_ref[...]) + 16.0 * bi_ref[...]
    s2s = dot_t(mu2_ref[...] * inv_f, wu_ref[...]) + 256.0 * bu_ref[...]
    x1 = dot_t(jnp.concatenate([h0, h1s], axis=1), w0_ref[...]) + b0_ref[...]
    agg = dot_t(jnp.concatenate([h1s, s2s], axis=1), w0_ref[...]) + 16.0 * b0_ref[...]
    out_ref[...] = dot_t(jnp.concatenate([x1, agg], axis=1), w1_ref[...]) + b1_ref[...]


def kernel(neighbors0, neighbors1, neighbors2, user_feat_idx, item_feat_idx,
           user_feat_emb, item_feat_emb, user_proj_w, user_proj_b,
           item_proj_w, item_proj_b, w0_w, w0_b, w1_w, w1_b):
    i32 = jnp.int32
    f32 = jnp.float32
    ufi = jnp.concatenate([
        user_feat_idx.astype(i32).reshape(-1),
        jnp.zeros(((_NU_PAD - user_feat_idx.shape[0]) * F,), i32),
    ])
    ifi = jnp.concatenate([
        item_feat_idx.astype(i32).reshape(-1),
        jnp.zeros(((_NI_PAD - item_feat_idx.shape[0]) * F,), i32),
    ])
    su, si = _embag2(user_feat_emb.astype(f32), ufi,
                     item_feat_emb.astype(f32), ifi)
    mu0, mi1, mu2 = _segsum3(su, si, neighbors0.astype(i32),
                             neighbors1.astype(i32), neighbors2.astype(i32))

    row = lambda v: v.astype(f32).reshape(1, D)
    return pl.pallas_call(
        _finale,
        out_shape=jax.ShapeDtypeStruct((B, D), jnp.float32),
    )(mu0, mi1, mu2,
      user_proj_w.astype(f32), row(user_proj_b),
      item_proj_w.astype(f32), row(item_proj_b),
      w0_w.astype(f32), row(w0_b),
      w1_w.astype(f32), row(w1_b))


# speedup vs baseline: 16.4561x; 1.2952x over previous
import functools

import jax
import jax.numpy as jnp
from jax import lax
from jax.experimental import pallas as pl
from jax.experimental.pallas import tpu as pltpu
from jax.experimental.pallas import tpu_sc as plsc

D = 64
NJ = D // 16
NC, NS = 2, 16
NW = NC * NS
F = 8
B = 2048

_NU_PAD = 100352
_NI_PAD = 50176
_UPW = _NU_PAD // NW
_IPW = _NI_PAD // NW
_CH = 56
_UCHUNKS = _UPW // _CH
_ICHUNKS = _IPW // _CH


def _wid():
    return lax.axis_index("s") * NC + lax.axis_index("c")


def _embag_stage(table, idx_v, out_hbm, row0, n_chunks, rows_bufs, gsems,
                 accs, ssem):
    rpc = _CH * F

    def issue(c, rb, gs):
        pltpu.make_async_copy(
            table.at[idx_v.at[pl.ds(c * rpc, rpc)]], rb, gs).start()

    def wait_gather(rb, gs):
        pltpu.make_async_copy(
            table.at[idx_v.at[pl.ds(0, rpc)]], rb, gs).wait()

    def wait_store():
        pltpu.make_async_copy(
            accs[0], out_hbm.at[pl.ds(row0, _CH)], ssem).wait()

    issue(0, rows_bufs[0], gsems[0])
    issue(1, rows_bufs[1], gsems[1])

    def quad(k, carry):
        @pl.when(k >= 1)
        def _():
            for _unused in range(4):
                wait_store()

        c0 = k * 4
        for q in range(4):
            c = c0 + q
            rb, gs, ab = rows_bufs[q % 2], gsems[q % 2], accs[q]
            wait_gather(rb, gs)

            def user_body(u, carry2):
                r = u * F
                for j in range(NJ):
                    s = rb[r, pl.ds(16 * j, 16)]
                    for f in range(1, F):
                        s = s + rb[r + f, pl.ds(16 * j, 16)]
                    ab[u, pl.ds(16 * j, 16)] = s
                return carry2

            lax.fori_loop(0, _CH, user_body, 0)

            @pl.when(c + 2 < n_chunks)
            def _():
                issue(c + 2, rb, gs)

            pltpu.make_async_copy(
                ab, out_hbm.at[pl.ds(row0 + c * _CH, _CH)], ssem).start()
        return carry

    lax.fori_loop(0, n_chunks // 4, quad, 0)
    for _unused in range(4):
        wait_store()


@functools.partial(
    pl.kernel,
    out_type=(jax.ShapeDtypeStruct((_NU_PAD, D), jnp.float32),
              jax.ShapeDtypeStruct((_NI_PAD, D), jnp.float32)),
    mesh=plsc.VectorSubcoreMesh(core_axis_name="c", subcore_axis_name="s"),
    compiler_params=pltpu.CompilerParams(use_tc_tiling_on_sc=False),
    scratch_types=[
        pltpu.VMEM((_UPW * F,), jnp.int32),
        pltpu.VMEM((_IPW * F,), jnp.int32),
        pltpu.VMEM((_CH * F, D), jnp.float32),
        pltpu.VMEM((_CH * F, D), jnp.float32),
        pltpu.VMEM((_CH, D), jnp.float32),
        pltpu.VMEM((_CH, D), jnp.float32),
        pltpu.VMEM((_CH, D), jnp.float32),
        pltpu.VMEM((_CH, D), jnp.float32),
        pltpu.VMEM_SHARED((3207, D), jnp.float32),
        pltpu.VMEM_SHARED((2094, D), jnp.float32),
        pltpu.SemaphoreType.DMA,
        pltpu.SemaphoreType.DMA,
        pltpu.SemaphoreType.DMA,
    ],
)
def _embag2(ue_hbm, ufi_hbm, ie_hbm, ifi_hbm, su_hbm, si_hbm,
            uidx_v, iidx_v, rows0, rows1, acc0, acc1, acc2, acc3,
            ue_sp, ie_sp, gsem0, gsem1, ssem):
    wid = _wid()
    u0 = wid * _UPW
    i0 = wid * _IPW

    @pl.when(lax.axis_index("s") == 0)
    def _():
        pltpu.sync_copy(ue_hbm, ue_sp)
        pltpu.sync_copy(ie_hbm, ie_sp)

    pltpu.sync_copy(ufi_hbm.at[pl.ds(u0 * F, _UPW * F)], uidx_v)
    pltpu.sync_copy(ifi_hbm.at[pl.ds(i0 * F, _IPW * F)], iidx_v)
    plsc.subcore_barrier()
    rows_bufs = (rows0, rows1)
    gsems = (gsem0, gsem1)
    accs = (acc0, acc1, acc2, acc3)
    _embag_stage(ue_sp, uidx_v, su_hbm, u0, _UCHUNKS, rows_bufs, gsems,
                 accs, ssem)
    _embag_stage(ie_sp, iidx_v, si_hbm, i0, _ICHUNKS, rows_bufs, gsems,
                 accs, ssem)


_RPW = B // NW
_G2 = 256
_G1 = 16
_SEG_ROWS = 512
_N2CH = _RPW * _G2 // _SEG_ROWS


@functools.partial(
    pl.kernel,
    out_type=(jax.ShapeDtypeStruct((B, D), jnp.float32),
              jax.ShapeDtypeStruct((B, D), jnp.float32),
              jax.ShapeDtypeStruct((B, D), jnp.float32)),
    mesh=plsc.VectorSubcoreMesh(core_axis_name="c", subcore_axis_name="s"),
    compiler_params=pltpu.CompilerParams(use_tc_tiling_on_sc=False),
    scratch_types=[
        pltpu.VMEM((_RPW * _G2,), jnp.int32),
        pltpu.VMEM((_RPW * _G1,), jnp.int32),
        pltpu.VMEM((_RPW,), jnp.int32),
        pltpu.VMEM((_SEG_ROWS, D), jnp.float32),
        pltpu.VMEM((_SEG_ROWS, D), jnp.float32),
        pltpu.VMEM((_RPW, D), jnp.float32),
        pltpu.VMEM((_RPW, D), jnp.float32),
        pltpu.VMEM((_RPW, D), jnp.float32),
        pltpu.SemaphoreType.DMA,
        pltpu.SemaphoreType.DMA,
        pltpu.SemaphoreType.DMA,
    ],
)
def _segsum3(su_hbm, si_hbm, n0_hbm, n1_hbm, n2_hbm,
             mu0_hbm, mi1_hbm, mu2_hbm,
             idx2_v, idx1_v, idx0_v, rows0, rows1, acc2, acc1, acc0,
             gsem0, gsem1, gsem2):
    wid = _wid()
    r0 = wid * _RPW
    pltpu.sync_copy(n0_hbm.at[pl.ds(r0, _RPW)], idx0_v)
    seg1 = pltpu.make_async_copy(su_hbm.at[idx0_v], acc0, gsem2)
    seg1.start()
    pltpu.sync_copy(n2_hbm.at[pl.ds(r0 * _G2, _RPW * _G2)], idx2_v)
    pltpu.sync_copy(n1_hbm.at[pl.ds(r0 * _G1, _RPW * _G1)], idx1_v)

    rows_bufs = (rows0, rows1)
    gsems = (gsem0, gsem1)

    def issue2(c, rb, gs):
        pltpu.make_async_copy(
            su_hbm.at[idx2_v.at[pl.ds(c * _SEG_ROWS, _SEG_ROWS)]], rb,
            gs).start()

    def wait_gather(table, rb, gs):
        pltpu.make_async_copy(
            table.at[idx2_v.at[pl.ds(0, _SEG_ROWS)]], rb, gs).wait()

    def reduce_group(rb, lo, group, acc, slot):
        def add_body(k, ss):
            return tuple(ss[j] + rb[lo + k, pl.ds(16 * j, 16)]
                         for j in range(NJ))

        init = tuple(rb[lo, pl.ds(16 * j, 16)] for j in range(NJ))
        ss = lax.fori_loop(1, group, add_body, init, unroll=4)
        for j in range(NJ):
            acc[slot, pl.ds(16 * j, 16)] = ss[j]

    issue2(0, rows0, gsem0)
    issue2(1, rows1, gsem1)

    def quad2(k, carry):
        c0 = k * 4
        for q in range(4):
            c = c0 + q
            rb, gs = rows_bufs[q % 2], gsems[q % 2]
            wait_gather(su_hbm, rb, gs)
            for lt in range(2):
                reduce_group(rb, lt * _G2, _G2, acc2, c * 2 + lt)

            @pl.when(c + 2 < _N2CH)
            def _():
                issue2(c + 2, rb, gs)
        return carry

    lax.fori_loop(0, _N2CH // 4, quad2, 0)

    for h in range(2):
        pltpu.make_async_copy(
            si_hbm.at[idx1_v.at[pl.ds(h * _SEG_ROWS, _SEG_ROWS)]],
            rows_bufs[h], gsems[h]).start()
    for h in range(2):
        wait_gather(si_hbm, rows_bufs[h], gsems[h])

        def root_body(t, carry2):
            reduce_group(rows_bufs[h], t * _G1, _G1, acc1,
                         h * (_SEG_ROWS // _G1) + t)
            return carry2

        lax.fori_loop(0, _SEG_ROWS // _G1, root_body, 0)

    pltpu.sync_copy(acc2, mu2_hbm.at[pl.ds(r0, _RPW)])
    pltpu.sync_copy(acc1, mi1_hbm.at[pl.ds(r0, _RPW)])
    seg1.wait()
    pltpu.sync_copy(acc0, mu0_hbm.at[pl.ds(r0, _RPW)])


def _finale(mu0_ref, mi1_ref, mu2_ref, wu_ref, bu_ref, wi_ref, bi_ref,
            w0_ref, b0_ref, w1_ref, b1_ref, out_ref):
    def dot_t(a, w):
        return lax.dot_general(a, w, (((1,), (1,)), ((), ())),
                               precision=lax.Precision.HIGHEST,
                               preferred_element_type=jnp.float32)

    inv_f = 1.0 / F
    h0 = dot_t(mu0_ref[...] * inv_f, wu_ref[...]) + bu_ref[...]
    h1s = dot_t(mi1_ref[...] * inv_f, wi_ref[...]) + 16.0 * bi_ref[...]
    s2s = dot_t(mu2_ref[...] * inv_f, wu_ref[...]) + 256.0 * bu_ref[...]
    x1 = dot_t(jnp.concatenate([h0, h1s], axis=1), w0_ref[...]) + b0_ref[...]
    agg = dot_t(jnp.concatenate([h1s, s2s], axis=1), w0_ref[...]) + 16.0 * b0_ref[...]
    out_ref[...] = dot_t(jnp.concatenate([x1, agg], axis=1), w1_ref[...]) + b1_ref[...]


def kernel(neighbors0, neighbors1, neighbors2, user_feat_idx, item_feat_idx,
           user_feat_emb, item_feat_emb, user_proj_w, user_proj_b,
           item_proj_w, item_proj_b, w0_w, w0_b, w1_w, w1_b):
    i32 = jnp.int32
    f32 = jnp.float32
    ufi = jnp.concatenate([
        user_feat_idx.astype(i32).reshape(-1),
        jnp.zeros(((_NU_PAD - user_feat_idx.shape[0]) * F,), i32),
    ])
    ifi = jnp.concatenate([
        item_feat_idx.astype(i32).reshape(-1),
        jnp.zeros(((_NI_PAD - item_feat_idx.shape[0]) * F,), i32),
    ])
    su, si = _embag2(user_feat_emb.astype(f32), ufi,
                     item_feat_emb.astype(f32), ifi)
    mu0, mi1, mu2 = _segsum3(su, si, neighbors0.astype(i32),
                             neighbors1.astype(i32), neighbors2.astype(i32))

    row = lambda v: v.astype(f32).reshape(1, D)
    return pl.pallas_call(
        _finale,
        out_shape=jax.ShapeDtypeStruct((B, D), jnp.float32),
    )(mu0, mi1, mu2,
      user_proj_w.astype(f32), row(user_proj_b),
      item_proj_w.astype(f32), row(item_proj_b),
      w0_w.astype(f32), row(w0_b),
      w1_w.astype(f32), row(w1_b))


# speedup vs baseline: 23.5676x; 1.4321x over previous
import functools

import jax
import jax.numpy as jnp
from jax import lax
from jax.experimental import pallas as pl
from jax.experimental.pallas import tpu as pltpu
from jax.experimental.pallas import tpu_sc as plsc

D = 64
NJ2 = D // 32
NC, NS = 2, 16
NW = NC * NS
F = 8
B = 2048
BF16 = jnp.bfloat16

_NU_PAD = 100352
_NI_PAD = 50176
_UPW = _NU_PAD // NW
_IPW = _NI_PAD // NW
_CH = 56
_UCHUNKS = _UPW // _CH
_ICHUNKS = _IPW // _CH


def _wid():
    return lax.axis_index("s") * NC + lax.axis_index("c")


def _embag_stage(table, idx_v, out_hbm, row0, n_chunks, rows_bufs, gsems,
                 accs, ssem):
    rpc = _CH * F

    def issue(c, rb, gs):
        pltpu.make_async_copy(
            table.at[idx_v.at[pl.ds(c * rpc, rpc)]], rb, gs).start()

    def wait_gather(rb, gs):
        pltpu.make_async_copy(
            table.at[idx_v.at[pl.ds(0, rpc)]], rb, gs).wait()

    def wait_store():
        pltpu.make_async_copy(
            accs[0], out_hbm.at[pl.ds(row0, _CH)], ssem).wait()

    issue(0, rows_bufs[0], gsems[0])
    issue(1, rows_bufs[1], gsems[1])

    def quad(k, carry):
        @pl.when(k >= 1)
        def _():
            for _unused in range(4):
                wait_store()

        c0 = k * 4
        for q in range(4):
            c = c0 + q
            rb, gs, ab = rows_bufs[q % 2], gsems[q % 2], accs[q]
            wait_gather(rb, gs)

            def user_body(u, carry2):
                r = u * F
                for j in range(NJ2):
                    s = rb[r, pl.ds(32 * j, 32)]
                    for f in range(1, F):
                        s = s + rb[r + f, pl.ds(32 * j, 32)]
                    ab[u, pl.ds(32 * j, 32)] = s
                return carry2

            lax.fori_loop(0, _CH, user_body, 0)

            @pl.when(c + 2 < n_chunks)
            def _():
                issue(c + 2, rb, gs)

            pltpu.make_async_copy(
                ab, out_hbm.at[pl.ds(row0 + c * _CH, _CH)], ssem).start()
        return carry

    lax.fori_loop(0, n_chunks // 4, quad, 0)
    for _unused in range(4):
        wait_store()


@functools.partial(
    pl.kernel,
    out_type=(jax.ShapeDtypeStruct((_NU_PAD, D), BF16),
              jax.ShapeDtypeStruct((_NI_PAD, D), BF16)),
    mesh=plsc.VectorSubcoreMesh(core_axis_name="c", subcore_axis_name="s"),
    compiler_params=pltpu.CompilerParams(use_tc_tiling_on_sc=False, needs_layout_passes=False),
    scratch_types=[
        pltpu.VMEM((_UPW * F,), jnp.int32),
        pltpu.VMEM((_IPW * F,), jnp.int32),
        pltpu.VMEM((_CH * F, D), BF16),
        pltpu.VMEM((_CH * F, D), BF16),
        pltpu.VMEM((_CH, D), BF16),
        pltpu.VMEM((_CH, D), BF16),
        pltpu.VMEM((_CH, D), BF16),
        pltpu.VMEM((_CH, D), BF16),
        pltpu.VMEM_SHARED((3207, D), BF16),
        pltpu.VMEM_SHARED((2094, D), BF16),
        pltpu.SemaphoreType.DMA,
        pltpu.SemaphoreType.DMA,
        pltpu.SemaphoreType.DMA,
    ],
)
def _embag2(ue_hbm, ufi_hbm, ie_hbm, ifi_hbm, su_hbm, si_hbm,
            uidx_v, iidx_v, rows0, rows1, acc0, acc1, acc2, acc3,
            ue_sp, ie_sp, gsem0, gsem1, ssem):
    wid = _wid()
    u0 = wid * _UPW
    i0 = wid * _IPW

    @pl.when(lax.axis_index("s") == 0)
    def _():
        pltpu.sync_copy(ue_hbm, ue_sp)
        pltpu.sync_copy(ie_hbm, ie_sp)

    pltpu.sync_copy(ufi_hbm.at[pl.ds(u0 * F, _UPW * F)], uidx_v)
    pltpu.sync_copy(ifi_hbm.at[pl.ds(i0 * F, _IPW * F)], iidx_v)
    plsc.subcore_barrier()
    rows_bufs = (rows0, rows1)
    gsems = (gsem0, gsem1)
    accs = (acc0, acc1, acc2, acc3)
    _embag_stage(ue_sp, uidx_v, su_hbm, u0, _UCHUNKS, rows_bufs, gsems,
                 accs, ssem)
    _embag_stage(ie_sp, iidx_v, si_hbm, i0, _ICHUNKS, rows_bufs, gsems,
                 accs, ssem)


_RPW = B // NW
_G2 = 256
_G1 = 16
_SEG_ROWS = 512
_N2CH = _RPW * _G2 // _SEG_ROWS


@functools.partial(
    pl.kernel,
    out_type=(jax.ShapeDtypeStruct((B, D), BF16),
              jax.ShapeDtypeStruct((B, D), BF16),
              jax.ShapeDtypeStruct((B, D), BF16)),
    mesh=plsc.VectorSubcoreMesh(core_axis_name="c", subcore_axis_name="s"),
    compiler_params=pltpu.CompilerParams(use_tc_tiling_on_sc=False, needs_layout_passes=False),
    scratch_types=[
        pltpu.VMEM((_RPW * _G2,), jnp.int32),
        pltpu.VMEM((_RPW * _G1,), jnp.int32),
        pltpu.VMEM((_RPW,), jnp.int32),
        pltpu.VMEM((_SEG_ROWS, D), BF16),
        pltpu.VMEM((_SEG_ROWS, D), BF16),
        pltpu.VMEM((_RPW, D), BF16),
        pltpu.VMEM((_RPW, D), BF16),
        pltpu.VMEM((_RPW, D), BF16),
        pltpu.SemaphoreType.DMA,
        pltpu.SemaphoreType.DMA,
        pltpu.SemaphoreType.DMA,
    ],
)
def _segsum3(su_hbm, si_hbm, n0_hbm, n1_hbm, n2_hbm,
             mu0_hbm, mi1_hbm, mu2_hbm,
             idx2_v, idx1_v, idx0_v, rows0, rows1, acc2, acc1, acc0,
             gsem0, gsem1, gsem2):
    wid = _wid()
    r0 = wid * _RPW
    pltpu.sync_copy(n0_hbm.at[pl.ds(r0, _RPW)], idx0_v)
    seg1 = pltpu.make_async_copy(su_hbm.at[idx0_v], acc0, gsem2)
    seg1.start()
    pltpu.sync_copy(n2_hbm.at[pl.ds(r0 * _G2, _RPW * _G2)], idx2_v)
    pltpu.sync_copy(n1_hbm.at[pl.ds(r0 * _G1, _RPW * _G1)], idx1_v)

    rows_bufs = (rows0, rows1)
    gsems = (gsem0, gsem1)

    def issue2(c, rb, gs):
        pltpu.make_async_copy(
            su_hbm.at[idx2_v.at[pl.ds(c * _SEG_ROWS, _SEG_ROWS)]], rb,
            gs).start()

    def wait_gather(table, rb, gs):
        pltpu.make_async_copy(
            table.at[idx2_v.at[pl.ds(0, _SEG_ROWS)]], rb, gs).wait()

    def reduce_group(rb, lo, group, acc, slot):
        def load(k):
            return [
                plsc.unpack(rb[lo + k, pl.ds(32 * j, 32)],
                            format=plsc.PackFormat.INTERLEAVED,
                            preferred_element_type=jnp.float32)
                for j in range(NJ2)
            ]

        def add_body(k, ss):
            cur = load(k)
            return tuple(
                (ss[j][0] + cur[j][0], ss[j][1] + cur[j][1])
                for j in range(NJ2)
            )

        init = tuple((p[0], p[1]) for p in load(0))
        ss = lax.fori_loop(1, group, add_body, init, unroll=4)
        for j in range(NJ2):
            acc[slot, pl.ds(32 * j, 32)] = plsc.pack(
                ss[j][0], ss[j][1], format=plsc.PackFormat.INTERLEAVED)

    issue2(0, rows0, gsem0)
    issue2(1, rows1, gsem1)

    def quad2(k, carry):
        c0 = k * 4
        for q in range(4):
            c = c0 + q
            rb, gs = rows_bufs[q % 2], gsems[q % 2]
            wait_gather(su_hbm, rb, gs)
            for lt in range(2):
                reduce_group(rb, lt * _G2, _G2, acc2, c * 2 + lt)

            @pl.when(c + 2 < _N2CH)
            def _():
                issue2(c + 2, rb, gs)
        return carry

    lax.fori_loop(0, _N2CH // 4, quad2, 0)

    for h in range(2):
        pltpu.make_async_copy(
            si_hbm.at[idx1_v.at[pl.ds(h * _SEG_ROWS, _SEG_ROWS)]],
            rows_bufs[h], gsems[h]).start()
    for h in range(2):
        wait_gather(si_hbm, rows_bufs[h], gsems[h])

        def root_body(t, carry2):
            reduce_group(rows_bufs[h], t * _G1, _G1, acc1,
                         h * (_SEG_ROWS // _G1) + t)
            return carry2

        lax.fori_loop(0, _SEG_ROWS // _G1, root_body, 0)

    pltpu.sync_copy(acc2, mu2_hbm.at[pl.ds(r0, _RPW)])
    pltpu.sync_copy(acc1, mi1_hbm.at[pl.ds(r0, _RPW)])
    seg1.wait()
    pltpu.sync_copy(acc0, mu0_hbm.at[pl.ds(r0, _RPW)])


def _finale(mu0_ref, mi1_ref, mu2_ref, wu_ref, bu_ref, wi_ref, bi_ref,
            w0_ref, b0_ref, w1_ref, b1_ref, out_ref):
    def dot_t(a, w):
        return lax.dot_general(a, w, (((1,), (1,)), ((), ())),
                               precision=lax.Precision.HIGHEST,
                               preferred_element_type=jnp.float32)

    inv_f = 1.0 / F
    mu0 = mu0_ref[...].astype(jnp.float32)
    mi1 = mi1_ref[...].astype(jnp.float32)
    mu2 = mu2_ref[...].astype(jnp.float32)
    h0 = dot_t(mu0 * inv_f, wu_ref[...]) + bu_ref[...]
    h1s = dot_t(mi1 * inv_f, wi_ref[...]) + 16.0 * bi_ref[...]
    s2s = dot_t(mu2 * inv_f, wu_ref[...]) + 256.0 * bu_ref[...]
    x1 = dot_t(jnp.concatenate([h0, h1s], axis=1), w0_ref[...]) + b0_ref[...]
    agg = dot_t(jnp.concatenate([h1s, s2s], axis=1), w0_ref[...]) + 16.0 * b0_ref[...]
    out_ref[...] = dot_t(jnp.concatenate([x1, agg], axis=1), w1_ref[...]) + b1_ref[...]


def kernel(neighbors0, neighbors1, neighbors2, user_feat_idx, item_feat_idx,
           user_feat_emb, item_feat_emb, user_proj_w, user_proj_b,
           item_proj_w, item_proj_b, w0_w, w0_b, w1_w, w1_b):
    i32 = jnp.int32
    f32 = jnp.float32
    ufi = jnp.concatenate([
        user_feat_idx.astype(i32).reshape(-1),
        jnp.zeros(((_NU_PAD - user_feat_idx.shape[0]) * F,), i32),
    ])
    ifi = jnp.concatenate([
        item_feat_idx.astype(i32).reshape(-1),
        jnp.zeros(((_NI_PAD - item_feat_idx.shape[0]) * F,), i32),
    ])
    su, si = _embag2(user_feat_emb.astype(BF16), ufi,
                     item_feat_emb.astype(BF16), ifi)
    mu0, mi1, mu2 = _segsum3(su, si, neighbors0.astype(i32),
                             neighbors1.astype(i32), neighbors2.astype(i32))

    row = lambda v: v.astype(f32).reshape(1, D)
    return pl.pallas_call(
        _finale,
        out_shape=jax.ShapeDtypeStruct((B, D), jnp.float32),
    )(mu0, mi1, mu2,
      user_proj_w.astype(f32), row(user_proj_b),
      item_proj_w.astype(f32), row(item_proj_b),
      w0_w.astype(f32), row(w0_b),
      w1_w.astype(f32), row(w1_b))


# speedup vs baseline: 24.4993x; 1.0395x over previous
import functools

import jax
import jax.numpy as jnp
from jax import lax
from jax.experimental import pallas as pl
from jax.experimental.pallas import tpu as pltpu
from jax.experimental.pallas import tpu_sc as plsc

D = 64
NJ2 = D // 32
NC, NS = 2, 16
NW = NC * NS
F = 8
B = 2048
BF16 = jnp.bfloat16

_NU_PAD = 100352
_NI_PAD = 50176
_UPW = _NU_PAD // NW
_IPW = _NI_PAD // NW
_CH = 56
_UCHUNKS = _UPW // _CH
_ICHUNKS = _IPW // _CH


def _wid():
    return lax.axis_index("s") * NC + lax.axis_index("c")


def _embag_stage(table, idx_v, out_hbm, row0, n_chunks, rows_bufs, gsems,
                 accs, ssem):
    rpc = _CH * F

    def issue(c, rb, gs):
        pltpu.make_async_copy(
            table.at[idx_v.at[pl.ds(c * rpc, rpc)]], rb, gs).start()

    def wait_gather(rb, gs):
        pltpu.make_async_copy(
            table.at[idx_v.at[pl.ds(0, rpc)]], rb, gs).wait()

    def wait_store():
        pltpu.make_async_copy(
            accs[0], out_hbm.at[pl.ds(row0, _CH)], ssem).wait()

    issue(0, rows_bufs[0], gsems[0])
    issue(1, rows_bufs[1], gsems[1])

    def quad(k, carry):
        @pl.when(k >= 1)
        def _():
            for _unused in range(4):
                wait_store()

        c0 = k * 4
        for q in range(4):
            c = c0 + q
            rb, gs, ab = rows_bufs[q % 2], gsems[q % 2], accs[q]
            wait_gather(rb, gs)

            def user_body(u, carry2):
                r = u * F
                for j in range(NJ2):
                    sl = pl.ds(32 * j, 32)
                    t01 = rb[r, sl] + rb[r + 1, sl]
                    t23 = rb[r + 2, sl] + rb[r + 3, sl]
                    t45 = rb[r + 4, sl] + rb[r + 5, sl]
                    t67 = rb[r + 6, sl] + rb[r + 7, sl]
                    ab[u, sl] = (t01 + t23) + (t45 + t67)
                return carry2

            lax.fori_loop(0, _CH, user_body, 0, unroll=2)

            @pl.when(c + 2 < n_chunks)
            def _():
                issue(c + 2, rb, gs)

            pltpu.make_async_copy(
                ab, out_hbm.at[pl.ds(row0 + c * _CH, _CH)], ssem).start()
        return carry

    lax.fori_loop(0, n_chunks // 4, quad, 0)
    for _unused in range(4):
        wait_store()


@functools.partial(
    pl.kernel,
    out_type=(jax.ShapeDtypeStruct((_NU_PAD, D), BF16),
              jax.ShapeDtypeStruct((_NI_PAD, D), BF16)),
    mesh=plsc.VectorSubcoreMesh(core_axis_name="c", subcore_axis_name="s"),
    compiler_params=pltpu.CompilerParams(use_tc_tiling_on_sc=False,
                                         needs_layout_passes=False,
                                         skip_device_barrier=True),
    scratch_types=[
        pltpu.VMEM((_UPW * F,), jnp.int32),
        pltpu.VMEM((_IPW * F,), jnp.int32),
        pltpu.VMEM((_CH * F, D), BF16),
        pltpu.VMEM((_CH * F, D), BF16),
        pltpu.VMEM((_CH, D), BF16),
        pltpu.VMEM((_CH, D), BF16),
        pltpu.VMEM((_CH, D), BF16),
        pltpu.VMEM((_CH, D), BF16),
        pltpu.VMEM_SHARED((3207, D), BF16),
        pltpu.VMEM_SHARED((2094, D), BF16),
        pltpu.SemaphoreType.DMA,
        pltpu.SemaphoreType.DMA,
        pltpu.SemaphoreType.DMA,
    ],
)
def _embag2(ue_hbm, ufi_hbm, ie_hbm, ifi_hbm, su_hbm, si_hbm,
            uidx_v, iidx_v, rows0, rows1, acc0, acc1, acc2, acc3,
            ue_sp, ie_sp, gsem0, gsem1, ssem):
    wid = _wid()
    u0 = wid * _UPW
    i0 = wid * _IPW

    @pl.when(lax.axis_index("s") == 0)
    def _():
        pltpu.sync_copy(ue_hbm, ue_sp)
        pltpu.sync_copy(ie_hbm, ie_sp)

    pltpu.sync_copy(ufi_hbm.at[pl.ds(u0 * F, _UPW * F)], uidx_v)
    pltpu.sync_copy(ifi_hbm.at[pl.ds(i0 * F, _IPW * F)], iidx_v)
    plsc.subcore_barrier()
    rows_bufs = (rows0, rows1)
    gsems = (gsem0, gsem1)
    accs = (acc0, acc1, acc2, acc3)
    _embag_stage(ue_sp, uidx_v, su_hbm, u0, _UCHUNKS, rows_bufs, gsems,
                 accs, ssem)
    _embag_stage(ie_sp, iidx_v, si_hbm, i0, _ICHUNKS, rows_bufs, gsems,
                 accs, ssem)


_RPW = B // NW
_G2 = 256
_G1 = 16
_SEG_ROWS = 512
_N2CH = _RPW * _G2 // _SEG_ROWS


@functools.partial(
    pl.kernel,
    out_type=(jax.ShapeDtypeStruct((B, D), BF16),
              jax.ShapeDtypeStruct((B, D), BF16),
              jax.ShapeDtypeStruct((B, D), BF16)),
    mesh=plsc.VectorSubcoreMesh(core_axis_name="c", subcore_axis_name="s"),
    compiler_params=pltpu.CompilerParams(use_tc_tiling_on_sc=False,
                                         needs_layout_passes=False,
                                         skip_device_barrier=True),
    scratch_types=[
        pltpu.VMEM((_RPW * _G2,), jnp.int32),
        pltpu.VMEM((_RPW * _G1,), jnp.int32),
        pltpu.VMEM((_RPW,), jnp.int32),
        pltpu.VMEM((_SEG_ROWS, D), BF16),
        pltpu.VMEM((_SEG_ROWS, D), BF16),
        pltpu.VMEM((_RPW, D), BF16),
        pltpu.VMEM((_RPW, D), BF16),
        pltpu.VMEM((_RPW, D), BF16),
        pltpu.SemaphoreType.DMA,
        pltpu.SemaphoreType.DMA,
        pltpu.SemaphoreType.DMA,
    ],
)
def _segsum3(su_hbm, si_hbm, n0_hbm, n1_hbm, n2_hbm,
             mu0_hbm, mi1_hbm, mu2_hbm,
             idx2_v, idx1_v, idx0_v, rows0, rows1, acc2, acc1, acc0,
             gsem0, gsem1, gsem2):
    wid = _wid()
    r0 = wid * _RPW
    pltpu.sync_copy(n0_hbm.at[pl.ds(r0, _RPW)], idx0_v)
    seg1 = pltpu.make_async_copy(su_hbm.at[idx0_v], acc0, gsem2)
    seg1.start()
    pltpu.sync_copy(n2_hbm.at[pl.ds(r0 * _G2, _RPW * _G2)], idx2_v)
    pltpu.sync_copy(n1_hbm.at[pl.ds(r0 * _G1, _RPW * _G1)], idx1_v)

    rows_bufs = (rows0, rows1)
    gsems = (gsem0, gsem1)

    def issue2(c, rb, gs):
        pltpu.make_async_copy(
            su_hbm.at[idx2_v.at[pl.ds(c * _SEG_ROWS, _SEG_ROWS)]], rb,
            gs).start()

    def wait_gather(table, rb, gs):
        pltpu.make_async_copy(
            table.at[idx2_v.at[pl.ds(0, _SEG_ROWS)]], rb, gs).wait()

    def reduce_group(rb, lo, group, acc, slot):
        def load(k):
            return [
                plsc.unpack(rb[lo + k, pl.ds(32 * j, 32)],
                            format=plsc.PackFormat.INTERLEAVED,
                            preferred_element_type=jnp.float32)
                for j in range(NJ2)
            ]

        def add_body(k, ss):
            cur = load(k)
            return tuple(
                (ss[j][0] + cur[j][0], ss[j][1] + cur[j][1])
                for j in range(NJ2)
            )

        init = tuple((p[0], p[1]) for p in load(0))
        ss = lax.fori_loop(1, group, add_body, init, unroll=4)
        for j in range(NJ2):
            acc[slot, pl.ds(32 * j, 32)] = plsc.pack(
                ss[j][0], ss[j][1], format=plsc.PackFormat.INTERLEAVED)

    issue2(0, rows0, gsem0)
    issue2(1, rows1, gsem1)

    def quad2(k, carry):
        c0 = k * 4
        for q in range(4):
            c = c0 + q
            rb, gs = rows_bufs[q % 2], gsems[q % 2]
            wait_gather(su_hbm, rb, gs)
            for lt in range(2):
                reduce_group(rb, lt * _G2, _G2, acc2, c * 2 + lt)

            @pl.when(c + 2 < _N2CH)
            def _():
                issue2(c + 2, rb, gs)
        return carry

    lax.fori_loop(0, _N2CH // 4, quad2, 0)

    for h in range(2):
        pltpu.make_async_copy(
            si_hbm.at[idx1_v.at[pl.ds(h * _SEG_ROWS, _SEG_ROWS)]],
            rows_bufs[h], gsems[h]).start()
    for h in range(2):
        wait_gather(si_hbm, rows_bufs[h], gsems[h])

        def root_body(t, carry2):
            reduce_group(rows_bufs[h], t * _G1, _G1, acc1,
                         h * (_SEG_ROWS // _G1) + t)
            return carry2

        lax.fori_loop(0, _SEG_ROWS // _G1, root_body, 0)

    pltpu.sync_copy(acc2, mu2_hbm.at[pl.ds(r0, _RPW)])
    pltpu.sync_copy(acc1, mi1_hbm.at[pl.ds(r0, _RPW)])
    seg1.wait()
    pltpu.sync_copy(acc0, mu0_hbm.at[pl.ds(r0, _RPW)])


def _finale(mu0_ref, mi1_ref, mu2_ref, wu_ref, bu_ref, wi_ref, bi_ref,
            w0_ref, b0_ref, w1_ref, b1_ref, out_ref):
    def dot_t(a, w):
        return lax.dot_general(a, w, (((1,), (1,)), ((), ())),
                               precision=lax.Precision.HIGHEST,
                               preferred_element_type=jnp.float32)

    inv_f = 1.0 / F
    mu0 = mu0_ref[...].astype(jnp.float32)
    mi1 = mi1_ref[...].astype(jnp.float32)
    mu2 = mu2_ref[...].astype(jnp.float32)
    h0 = dot_t(mu0 * inv_f, wu_ref[...]) + bu_ref[...]
    h1s = dot_t(mi1 * inv_f, wi_ref[...]) + 16.0 * bi_ref[...]
    s2s = dot_t(mu2 * inv_f, wu_ref[...]) + 256.0 * bu_ref[...]
    x1 = dot_t(jnp.concatenate([h0, h1s], axis=1), w0_ref[...]) + b0_ref[...]
    agg = dot_t(jnp.concatenate([h1s, s2s], axis=1), w0_ref[...]) + 16.0 * b0_ref[...]
    out_ref[...] = dot_t(jnp.concatenate([x1, agg], axis=1), w1_ref[...]) + b1_ref[...]


def kernel(neighbors0, neighbors1, neighbors2, user_feat_idx, item_feat_idx,
           user_feat_emb, item_feat_emb, user_proj_w, user_proj_b,
           item_proj_w, item_proj_b, w0_w, w0_b, w1_w, w1_b):
    i32 = jnp.int32
    f32 = jnp.float32
    ufi = jnp.concatenate([
        user_feat_idx.astype(i32).reshape(-1),
        jnp.zeros(((_NU_PAD - user_feat_idx.shape[0]) * F,), i32),
    ])
    ifi = jnp.concatenate([
        item_feat_idx.astype(i32).reshape(-1),
        jnp.zeros(((_NI_PAD - item_feat_idx.shape[0]) * F,), i32),
    ])
    su, si = _embag2(user_feat_emb.astype(BF16), ufi,
                     item_feat_emb.astype(BF16), ifi)
    mu0, mi1, mu2 = _segsum3(su, si, neighbors0.astype(i32),
                             neighbors1.astype(i32), neighbors2.astype(i32))

    row = lambda v: v.astype(f32).reshape(1, D)
    return pl.pallas_call(
        _finale,
        out_shape=jax.ShapeDtypeStruct((B, D), jnp.float32),
    )(mu0, mi1, mu2,
      user_proj_w.astype(f32), row(user_proj_b),
      item_proj_w.astype(f32), row(item_proj_b),
      w0_w.astype(f32), row(w0_b),
      w1_w.astype(f32), row(w1_b))


# speedup vs baseline: 27.1185x; 1.1069x over previous
import functools

import jax
import jax.numpy as jnp
from jax import lax
from jax.experimental import pallas as pl
from jax.experimental.pallas import tpu as pltpu
from jax.experimental.pallas import tpu_sc as plsc

D = 64
NJ2 = D // 32
NC, NS = 2, 16
NW = NC * NS
F = 8
B = 2048
BF16 = jnp.bfloat16

NU = 100000
NI = 50000
_UPW = 3136
_IPW = 1568
_CH = 56
_UCHUNKS = _UPW // _CH
_ICHUNKS = _IPW // _CH

_SC_PARAMS = pltpu.CompilerParams(use_tc_tiling_on_sc=False,
                                  needs_layout_passes=False,
                                  skip_device_barrier=True)


def _wid():
    return lax.axis_index("s") * NC + lax.axis_index("c")


def _embag_stage(table, idx_v, out_hbm, row0, n_chunks, rows_bufs, gsems,
                 accs, ssem):
    rpc = _CH * F

    def issue(c, rb, gs):
        pltpu.make_async_copy(
            table.at[idx_v.at[pl.ds(c * rpc, rpc)]], rb, gs).start()

    def wait_gather(rb, gs):
        pltpu.make_async_copy(
            table.at[idx_v.at[pl.ds(0, rpc)]], rb, gs).wait()

    def wait_store():
        pltpu.make_async_copy(
            accs[0], out_hbm.at[pl.ds(row0, _CH)], ssem).wait()

    issue(0, rows_bufs[0], gsems[0])
    issue(1, rows_bufs[1], gsems[1])

    def quad(k, carry):
        @pl.when(k >= 1)
        def _():
            for _unused in range(4):
                wait_store()

        c0 = k * 4
        for q in range(4):
            c = c0 + q
            rb, gs, ab = rows_bufs[q % 2], gsems[q % 2], accs[q]
            wait_gather(rb, gs)

            def user_body(u, carry2):
                r = u * F
                for j in range(NJ2):
                    sl = pl.ds(32 * j, 32)
                    t01 = rb[r, sl] + rb[r + 1, sl]
                    t23 = rb[r + 2, sl] + rb[r + 3, sl]
                    t45 = rb[r + 4, sl] + rb[r + 5, sl]
                    t67 = rb[r + 6, sl] + rb[r + 7, sl]
                    ab[u, sl] = (t01 + t23) + (t45 + t67)
                return carry2

            lax.fori_loop(0, _CH, user_body, 0, unroll=2)

            @pl.when(c + 2 < n_chunks)
            def _():
                issue(c + 2, rb, gs)

            pltpu.make_async_copy(
                ab, out_hbm.at[pl.ds(row0 + c * _CH, _CH)], ssem).start()
        return carry

    lax.fori_loop(0, n_chunks // 4, quad, 0)
    for _unused in range(4):
        wait_store()


def _make_embag(n_rows, tab_rows, per_w, n_chunks):
    @functools.partial(
        pl.kernel,
        out_type=jax.ShapeDtypeStruct((n_rows, D), BF16),
        mesh=plsc.VectorSubcoreMesh(core_axis_name="c", subcore_axis_name="s"),
        compiler_params=_SC_PARAMS,
        scratch_types=[
            pltpu.VMEM((per_w * F,), jnp.int32),
            pltpu.VMEM((_CH * F, D), BF16),
            pltpu.VMEM((_CH * F, D), BF16),
            pltpu.VMEM((_CH, D), BF16),
            pltpu.VMEM((_CH, D), BF16),
            pltpu.VMEM((_CH, D), BF16),
            pltpu.VMEM((_CH, D), BF16),
            pltpu.VMEM_SHARED((tab_rows, D), BF16),
            pltpu.SemaphoreType.DMA,
            pltpu.SemaphoreType.DMA,
            pltpu.SemaphoreType.DMA,
        ],
    )
    def embag(tab_hbm, fidx_hbm, out_hbm,
              idx_v, rows0, rows1, acc0, acc1, acc2, acc3,
              tab_sp, gsem0, gsem1, ssem):
        row0 = lax.min(_wid() * per_w, n_rows - per_w)

        @pl.when(lax.axis_index("s") == 0)
        def _():
            pltpu.sync_copy(tab_hbm, tab_sp)

        pltpu.sync_copy(fidx_hbm.at[pl.ds(row0 * F, per_w * F)], idx_v)
        plsc.subcore_barrier()
        _embag_stage(tab_sp, idx_v, out_hbm, row0, n_chunks,
                     (rows0, rows1), (gsem0, gsem1),
                     (acc0, acc1, acc2, acc3), ssem)

    return embag


_embag_items = _make_embag(NI, 2094, _IPW, _ICHUNKS)
_embag_users = _make_embag(NU, 3207, _UPW, _UCHUNKS)


_RPW = B // NW
_G2 = 256
_G1 = 16
_SEG_ROWS = 512
_N2CH = _RPW * _G2 // _SEG_ROWS


@functools.partial(
    pl.kernel,
    out_type=(jax.ShapeDtypeStruct((B, D), BF16),
              jax.ShapeDtypeStruct((B, D), BF16),
              jax.ShapeDtypeStruct((B, D), BF16)),
    mesh=plsc.VectorSubcoreMesh(core_axis_name="c", subcore_axis_name="s"),
    compiler_params=_SC_PARAMS,
    scratch_types=[
        pltpu.VMEM((_RPW * _G2,), jnp.int32),
        pltpu.VMEM((_RPW * _G1,), jnp.int32),
        pltpu.VMEM((_RPW,), jnp.int32),
        pltpu.VMEM((_SEG_ROWS, D), BF16),
        pltpu.VMEM((_SEG_ROWS, D), BF16),
        pltpu.VMEM((_RPW, D), BF16),
        pltpu.VMEM((_RPW, D), BF16),
        pltpu.VMEM((_RPW, D), BF16),
        pltpu.SemaphoreType.DMA,
        pltpu.SemaphoreType.DMA,
        pltpu.SemaphoreType.DMA,
    ],
)
def _segsum3(su_hbm, si_hbm, n0_hbm, n1_hbm, n2_hbm,
             mu0_hbm, mi1_hbm, mu2_hbm,
             idx2_v, idx1_v, idx0_v, rows0, rows1, acc2, acc1, acc0,
             gsem0, gsem1, gsem2):
    wid = _wid()
    r0 = wid * _RPW
    pltpu.sync_copy(n0_hbm.at[pl.ds(r0, _RPW)], idx0_v)
    seg1 = pltpu.make_async_copy(su_hbm.at[idx0_v], acc0, gsem2)
    seg1.start()
    pltpu.sync_copy(n2_hbm.at[pl.ds(r0 * _G2, _RPW * _G2)], idx2_v)
    pltpu.sync_copy(n1_hbm.at[pl.ds(r0 * _G1, _RPW * _G1)], idx1_v)

    rows_bufs = (rows0, rows1)
    gsems = (gsem0, gsem1)

    def issue2(c, rb, gs):
        pltpu.make_async_copy(
            su_hbm.at[idx2_v.at[pl.ds(c * _SEG_ROWS, _SEG_ROWS)]], rb,
            gs).start()

    def wait_gather(table, rb, gs):
        pltpu.make_async_copy(
            table.at[idx2_v.at[pl.ds(0, _SEG_ROWS)]], rb, gs).wait()

    def reduce_group(rb, lo, group, acc, slot):
        def load(k):
            return [
                plsc.unpack(rb[lo + k, pl.ds(32 * j, 32)],
                            format=plsc.PackFormat.INTERLEAVED,
                            preferred_element_type=jnp.float32)
                for j in range(NJ2)
            ]

        def add_body(k, ss):
            cur = load(k)
            return tuple(
                (ss[j][0] + cur[j][0], ss[j][1] + cur[j][1])
                for j in range(NJ2)
            )

        init = tuple((p[0], p[1]) for p in load(0))
        ss = lax.fori_loop(1, group, add_body, init, unroll=4)
        for j in range(NJ2):
            acc[slot, pl.ds(32 * j, 32)] = plsc.pack(
                ss[j][0], ss[j][1], format=plsc.PackFormat.INTERLEAVED)

    issue2(0, rows0, gsem0)
    issue2(1, rows1, gsem1)

    def quad2(k, carry):
        c0 = k * 4
        for q in range(4):
            c = c0 + q
            rb, gs = rows_bufs[q % 2], gsems[q % 2]
            wait_gather(su_hbm, rb, gs)
            for lt in range(2):
                reduce_group(rb, lt * _G2, _G2, acc2, c * 2 + lt)

            @pl.when(c + 2 < _N2CH)
            def _():
                issue2(c + 2, rb, gs)
        return carry

    lax.fori_loop(0, _N2CH // 4, quad2, 0)

    for h in range(2):
        pltpu.make_async_copy(
            si_hbm.at[idx1_v.at[pl.ds(h * _SEG_ROWS, _SEG_ROWS)]],
            rows_bufs[h], gsems[h]).start()
    for h in range(2):
        wait_gather(si_hbm, rows_bufs[h], gsems[h])

        def root_body(t, carry2):
            reduce_group(rows_bufs[h], t * _G1, _G1, acc1,
                         h * (_SEG_ROWS // _G1) + t)
            return carry2

        lax.fori_loop(0, _SEG_ROWS // _G1, root_body, 0)

    pltpu.sync_copy(acc2, mu2_hbm.at[pl.ds(r0, _RPW)])
    pltpu.sync_copy(acc1, mi1_hbm.at[pl.ds(r0, _RPW)])
    seg1.wait()
    pltpu.sync_copy(acc0, mu0_hbm.at[pl.ds(r0, _RPW)])


def _finale(mu0_ref, mi1_ref, mu2_ref, wu_ref, bu_ref, wi_ref, bi_ref,
            w0_ref, b0_ref, w1_ref, b1_ref, out_ref):
    def dot_t(a, w):
        return lax.dot_general(a, w, (((1,), (1,)), ((), ())),
                               precision=lax.Precision.HIGHEST,
                               preferred_element_type=jnp.float32)

    inv_f = 1.0 / F
    mu0 = mu0_ref[...].astype(jnp.float32)
    mi1 = mi1_ref[...].astype(jnp.float32)
    mu2 = mu2_ref[...].astype(jnp.float32)
    h0 = dot_t(mu0 * inv_f, wu_ref[...]) + bu_ref[...]
    h1s = dot_t(mi1 * inv_f, wi_ref[...]) + 16.0 * bi_ref[...]
    s2s = dot_t(mu2 * inv_f, wu_ref[...]) + 256.0 * bu_ref[...]
    x1 = dot_t(jnp.concatenate([h0, h1s], axis=1), w0_ref[...]) + b0_ref[...]
    agg = dot_t(jnp.concatenate([h1s, s2s], axis=1), w0_ref[...]) + 16.0 * b0_ref[...]
    out_ref[...] = dot_t(jnp.concatenate([x1, agg], axis=1), w1_ref[...]) + b1_ref[...]


def kernel(neighbors0, neighbors1, neighbors2, user_feat_idx, item_feat_idx,
           user_feat_emb, item_feat_emb, user_proj_w, user_proj_b,
           item_proj_w, item_proj_b, w0_w, w0_b, w1_w, w1_b):
    i32 = jnp.int32
    f32 = jnp.float32
    ifi = item_feat_idx.astype(i32).reshape(-1)
    ufi = user_feat_idx.astype(i32).reshape(-1)
    si = _embag_items(item_feat_emb.astype(BF16), ifi)
    su = _embag_users(user_feat_emb.astype(BF16), ufi)
    mu0, mi1, mu2 = _segsum3(su, si, neighbors0.astype(i32),
                             neighbors1.astype(i32), neighbors2.astype(i32))

    row = lambda v: v.astype(f32).reshape(1, D)
    return pl.pallas_call(
        _finale,
        out_shape=jax.ShapeDtypeStruct((B, D), jnp.float32),
    )(mu0, mi1, mu2,
      user_proj_w.astype(f32), row(user_proj_b),
      item_proj_w.astype(f32), row(item_proj_b),
      w0_w.astype(f32), row(w0_b),
      w1_w.astype(f32), row(w1_b))
